# Initial kernel scaffold; baseline (speedup 1.0000x reference)
import jax
import jax.numpy as jnp
from jax.experimental import pallas as pl


def kernel(x, fc_w, fc_b, percentile):
    raise NotImplementedError("write your pallas kernel here")


# speedup vs baseline: 4.6210x; 4.6210x over previous
import functools

import jax
import jax.numpy as jnp
from jax import lax
from jax.experimental import pallas as pl
from jax.experimental.pallas import tpu as pltpu


def _body(k_ref, x_ref, w_ref, b_ref, o_ref):
    xv = x_ref[...]
    kk = k_ref[0]
    kf = kk.astype(jnp.float32)
    xb = lax.bitcast_convert_type(xv, jnp.int32)
    bsz = xv.shape[0]

    s1 = jnp.sum(xv, axis=1, keepdims=True)

    lo0 = jnp.zeros((bsz, 1), jnp.int32)
    hi0 = jnp.full((bsz, 1), 0x7F800000, jnp.int32)

    def bisect(_, carry):
        lo, hi = carry
        mid = lo + ((hi - lo) >> 1)
        cnt = jnp.sum((xb >= mid).astype(jnp.int32), axis=1, keepdims=True)
        ge = cnt >= kk
        return jnp.where(ge, mid, lo), jnp.where(ge, hi, mid)

    lo, _ = lax.fori_loop(0, 31, bisect, (lo0, hi0))
    vk = lax.bitcast_convert_type(lo, jnp.float32)

    gt = xv > vk
    cnt_gt = jnp.sum(gt.astype(jnp.float32), axis=1, keepdims=True)
    sum_gt = jnp.sum(jnp.where(gt, xv, 0.0), axis=1, keepdims=True)
    s2 = sum_gt + (kf - cnt_gt) * vk
    scale = jnp.exp(s1 / s2)

    y = lax.dot_general(xv, w_ref[...], (((1,), (1,)), ((), ())),
                        preferred_element_type=jnp.float32)
    o_ref[...] = y * scale + b_ref[...]


def kernel(x, fc_w, fc_b, percentile):
    b, c, h, w = x.shape
    n = c * h * w
    x2 = x.reshape(b, n)
    nc = fc_w.shape[0]
    kk = (n - jnp.round(n * percentile / 100.0)).astype(jnp.int32).reshape(1)
    out = pl.pallas_call(
        _body,
        out_shape=jax.ShapeDtypeStruct((b, nc), jnp.float32),
        in_specs=[
            pl.BlockSpec(memory_space=pltpu.SMEM),
            pl.BlockSpec(memory_space=pltpu.VMEM),
            pl.BlockSpec(memory_space=pltpu.VMEM),
            pl.BlockSpec(memory_space=pltpu.VMEM),
        ],
    )(kk, x2, fc_w, fc_b.reshape(1, nc))
    return out
